# baseline (speedup 1.0000x reference)
import functools
import jax
import jax.numpy as jnp
from jax import lax
from jax.experimental import pallas as pl
from jax.experimental.pallas import tpu as pltpu, tpu_sc as plsc


def _make_sc(B, R, D, dtype):
    info = plsc.get_sparse_core_info()
    NC, NS = info.num_cores, info.num_subcores
    NW = NC * NS
    rows_per_w = R // NW

    mesh = plsc.VectorSubcoreMesh(core_axis_name="c", subcore_axis_name="s")

    @functools.partial(
        pl.kernel,
        mesh=mesh,
        out_type=jax.ShapeDtypeStruct((B, R, D), dtype),
        scratch_types=[pltpu.SemaphoreType.DMA],
    )
    def k(wpe_hbm, out_hbm, sem):
        wid = lax.axis_index("s") * NC + lax.axis_index("c")
        base = wid * rows_per_w
        copies = [
            pltpu.async_copy(
                wpe_hbm.at[pl.ds(base, rows_per_w)],
                out_hbm.at[b, pl.ds(base, rows_per_w)],
                sem,
            )
            for b in range(B)
        ]
        for c in copies:
            c.wait()

    return k


def kernel(x, wpe):
    B, S = x.shape
    R, D = wpe.shape
    return _make_sc(B, R, D, wpe.dtype)(wpe)


# speedup vs baseline: 54.0501x; 54.0501x over previous
import functools
import jax
import jax.numpy as jnp
from jax import lax
from jax.experimental import pallas as pl
from jax.experimental.pallas import tpu as pltpu, tpu_sc as plsc


def _make_sc(B, R, D, dtype):
    info = plsc.get_sparse_core_info()
    NC, NS = info.num_cores, info.num_subcores
    NW = NC * NS
    rows_per_w = R // NW
    CH = 32
    n_ch = rows_per_w // CH

    mesh = plsc.VectorSubcoreMesh(core_axis_name="c", subcore_axis_name="s")

    @functools.partial(
        pl.kernel,
        mesh=mesh,
        out_type=jax.ShapeDtypeStruct((B, R, D), dtype),
        scratch_types=[
            pltpu.VMEM((CH, D), dtype),
            pltpu.VMEM((CH, D), dtype),
            pltpu.SemaphoreType.DMA,
            pltpu.SemaphoreType.DMA,
            pltpu.SemaphoreType.DMA,
            pltpu.SemaphoreType.DMA,
        ],
    )
    def k(wpe_hbm, out_hbm, buf0, buf1, rs0, rs1, ws0, ws1):
        wid = lax.axis_index("s") * NC + lax.axis_index("c")
        base = wid * rows_per_w
        bufs = (buf0, buf1)
        rsems = (rs0, rs1)
        wsems = (ws0, ws1)

        reads = [None, None]
        writes = [[], []]
        reads[0] = pltpu.async_copy(wpe_hbm.at[pl.ds(base, CH)], bufs[0], rsems[0])
        for kk in range(n_ch):
            cur = kk % 2
            nxt = (kk + 1) % 2
            if kk + 1 < n_ch:
                r0n = base + (kk + 1) * CH
                for w in writes[nxt]:
                    w.wait()
                writes[nxt] = []
                reads[nxt] = pltpu.async_copy(
                    wpe_hbm.at[pl.ds(r0n, CH)], bufs[nxt], rsems[nxt]
                )
            reads[cur].wait()
            r0 = base + kk * CH
            writes[cur] = [
                pltpu.async_copy(
                    bufs[cur], out_hbm.at[b, pl.ds(r0, CH)], wsems[cur]
                )
                for b in range(B)
            ]
        for side in writes:
            for w in side:
                w.wait()

    return k


def kernel(x, wpe):
    B, S = x.shape
    R, D = wpe.shape
    return _make_sc(B, R, D, wpe.dtype)(wpe)


# speedup vs baseline: 55.8326x; 1.0330x over previous
import functools
import jax
import jax.numpy as jnp
from jax import lax
from jax.experimental import pallas as pl
from jax.experimental.pallas import tpu as pltpu, tpu_sc as plsc


def _make_sc(B, R, D, dtype):
    info = plsc.get_sparse_core_info()
    NC, NS = info.num_cores, info.num_subcores
    NW = NC * NS
    rows_per_w = R // NW
    CH = 64
    n_ch = rows_per_w // CH

    mesh = plsc.VectorSubcoreMesh(core_axis_name="c", subcore_axis_name="s")

    @functools.partial(
        pl.kernel,
        mesh=mesh,
        out_type=jax.ShapeDtypeStruct((B, R, D), dtype),
        scratch_types=[pltpu.VMEM((CH, D), dtype)],
    )
    def k(wpe_hbm, out_hbm, buf):
        wid = lax.axis_index("s") * NC + lax.axis_index("c")
        base = wid * rows_per_w
        for kk in range(n_ch):
            r0 = base + kk * CH
            pltpu.sync_copy(wpe_hbm.at[pl.ds(r0, CH)], buf)
            for b in range(B):
                pltpu.sync_copy(buf, out_hbm.at[b, pl.ds(r0, CH)])

    return k


def kernel(x, wpe):
    B, S = x.shape
    R, D = wpe.shape
    return _make_sc(B, R, D, wpe.dtype)(wpe)
